# baseline (speedup 1.0000x reference)
import functools

import jax
import jax.numpy as jnp
from jax import lax
from jax.experimental import pallas as pl
from jax.experimental.pallas import tpu as pltpu
from jax.experimental.pallas import tpu_sc as plsc

B, S = 16384, 200
TAB = 1000000
NC, NS = 2, 16
NW = NC * NS
ROWS_W = B // NW
RCHUNK = 32
RCHUNKS = ROWS_W // RCHUNK
RSTRIDE = 208
FLAT = RCHUNK * RSTRIDE
MCOLS = tuple(range(0, 192, 16)) + (184,)
PIECE = 10000
PIECES = TAB // PIECE

_mesh = plsc.VectorSubcoreMesh(core_axis_name="c", subcore_axis_name="s")


@functools.partial(
    pl.kernel,
    mesh=_mesh,
    out_type=jax.ShapeDtypeStruct((B, S), jnp.float32),
    scratch_types=[
        pltpu.VMEM_SHARED((TAB,), jnp.float32),
        pltpu.VMEM((PIECE,), jnp.float32),
        pltpu.VMEM((RCHUNK, S), jnp.int32),
        pltpu.VMEM((RCHUNK, S), jnp.float32),
        pltpu.VMEM((FLAT,), jnp.int32),
        pltpu.VMEM((FLAT,), jnp.float32),
        pltpu.SemaphoreType.DMA,
    ],
)
def _gather_sc(idx_hbm, u_hbm, out_hbm, u_sp, bounce, tmp_i, tmp_o,
               idx_f, out_f, sem):
    sid = lax.axis_index("s")
    wid = sid * NC + lax.axis_index("c")

    @pl.loop(0, RCHUNK)
    def _pad(r):
        base = lax.iota(jnp.int32, 16) * 8 + r * 128
        idx_f[pl.ds(r * RSTRIDE + 192, 16)] = base

    for j in range((PIECES + NS - 1) // NS):
        piece = sid + NS * j

        @pl.when(piece < PIECES)
        def _stage():
            off = piece * PIECE
            pltpu.sync_copy(u_hbm.at[pl.ds(off, PIECE)], bounce)
            pltpu.sync_copy(bounce, u_sp.at[pl.ds(off, PIECE)])

    plsc.subcore_barrier()

    r00 = wid * ROWS_W
    for k in range(RCHUNKS):
        r0 = r00 + k * RCHUNK
        pltpu.sync_copy(idx_hbm.at[pl.ds(r0, RCHUNK), :], tmp_i)

        @pl.loop(0, RCHUNK)
        def _pack(r):
            for c in MCOLS:
                idx_f[pl.ds(r * RSTRIDE + c, 16)] = tmp_i[r, pl.ds(c, 16)]

        pltpu.async_copy(u_sp.at[idx_f], out_f, sem).wait()

        @pl.loop(0, RCHUNK)
        def _unpack(r):
            for c in MCOLS:
                tmp_o[r, pl.ds(c, 16)] = out_f[pl.ds(r * RSTRIDE + c, 16)]

        pltpu.sync_copy(tmp_o, out_hbm.at[pl.ds(r0, RCHUNK), :])


def kernel(idx, u):
    return _gather_sc(idx, u)


# speedup vs baseline: 1.3201x; 1.3201x over previous
import functools

import jax
import jax.numpy as jnp
from jax import lax
from jax.experimental import pallas as pl
from jax.experimental.pallas import tpu as pltpu
from jax.experimental.pallas import tpu_sc as plsc

B, S = 16384, 200
TAB = 1000000
NC, NS = 2, 16
NW = NC * NS
ROWS_W = B // NW
RCHUNK = 32
RCHUNKS = ROWS_W // RCHUNK
RSTRIDE = 208
FLAT = RCHUNK * RSTRIDE
MCOLS = tuple(range(0, 192, 16)) + (184,)
PIECE = 10000
PIECES = TAB // PIECE

_mesh = plsc.VectorSubcoreMesh(core_axis_name="c", subcore_axis_name="s")


@functools.partial(
    pl.kernel,
    mesh=_mesh,
    out_type=jax.ShapeDtypeStruct((B, S), jnp.float32),
    scratch_types=[
        pltpu.VMEM_SHARED((TAB,), jnp.float32),
        pltpu.VMEM((PIECE,), jnp.float32),
        pltpu.VMEM((RCHUNK, S), jnp.int32),
        pltpu.VMEM((RCHUNK, S), jnp.float32),
        pltpu.VMEM((FLAT,), jnp.int32),
        pltpu.VMEM((FLAT,), jnp.float32),
        pltpu.SemaphoreType.DMA,
    ],
)
def _gather_sc(idx_hbm, u_hbm, out_hbm, u_sp, bounce, tmp_i, tmp_o,
               idx_f, out_f, sem):
    sid = lax.axis_index("s")
    wid = sid * NC + lax.axis_index("c")

    @pl.loop(0, RCHUNK)
    def _pad(r):
        base = lax.iota(jnp.int32, 16) * 8 + r * 128
        idx_f[pl.ds(r * RSTRIDE + 192, 16)] = base

    for j in range((PIECES + NS - 1) // NS):
        piece = sid + NS * j

        @pl.when(piece < PIECES)
        def _stage():
            off = piece * PIECE
            pltpu.sync_copy(u_hbm.at[pl.ds(off, PIECE)], bounce)
            pltpu.sync_copy(bounce, u_sp.at[pl.ds(off, PIECE)])

    plsc.subcore_barrier()

    r00 = wid * ROWS_W

    @pl.loop(0, RCHUNKS)
    def _slab(k):
        r0 = r00 + k * RCHUNK
        pltpu.sync_copy(idx_hbm.at[pl.ds(r0, RCHUNK), :], tmp_i)

        for r in range(RCHUNK):
            for c in MCOLS:
                idx_f[pl.ds(r * RSTRIDE + c, 16)] = tmp_i[r, pl.ds(c, 16)]

        pltpu.async_copy(u_sp.at[idx_f], out_f, sem).wait()

        for r in range(RCHUNK):
            for c in MCOLS:
                tmp_o[r, pl.ds(c, 16)] = out_f[pl.ds(r * RSTRIDE + c, 16)]

        pltpu.sync_copy(tmp_o, out_hbm.at[pl.ds(r0, RCHUNK), :])


def kernel(idx, u):
    return _gather_sc(idx, u)


# speedup vs baseline: 1.6935x; 1.2828x over previous
import functools

import jax
import jax.numpy as jnp
from jax import lax
from jax.experimental import pallas as pl
from jax.experimental.pallas import tpu as pltpu
from jax.experimental.pallas import tpu_sc as plsc

B, S = 16384, 200
TAB = 1000000
NC, NS = 2, 16
NW = NC * NS
ROWS_W = B // NW
RCHUNK = 32
RCHUNKS = ROWS_W // RCHUNK
RSTRIDE = 208
FLAT = RCHUNK * RSTRIDE
MCOLS = tuple(range(0, 192, 16)) + (184,)
PIECE = 8000
PIECES = TAB // PIECE

_mesh = plsc.VectorSubcoreMesh(core_axis_name="c", subcore_axis_name="s")


@functools.partial(
    pl.kernel,
    mesh=_mesh,
    out_type=jax.ShapeDtypeStruct((B, S), jnp.float32),
    scratch_types=[
        pltpu.VMEM_SHARED((TAB,), jnp.float32),
        pltpu.VMEM((PIECE,), jnp.float32),
        pltpu.VMEM((RCHUNK, S), jnp.int32),
        pltpu.VMEM((RCHUNK, S), jnp.int32),
        pltpu.VMEM((RCHUNK, S), jnp.float32),
        pltpu.VMEM((RCHUNK, S), jnp.float32),
        pltpu.VMEM((FLAT,), jnp.int32),
        pltpu.VMEM((FLAT,), jnp.int32),
        pltpu.VMEM((FLAT,), jnp.float32),
        pltpu.VMEM((FLAT,), jnp.float32),
        pltpu.SemaphoreType.DMA,
        pltpu.SemaphoreType.DMA,
        pltpu.SemaphoreType.DMA,
        pltpu.SemaphoreType.DMA,
        pltpu.SemaphoreType.DMA,
        pltpu.SemaphoreType.DMA,
    ],
)
def _gather_sc(idx_hbm, u_hbm, out_hbm, u_sp, bounce,
               ti0, ti1, to0, to1, if0, if1, of0, of1,
               sin0, sin1, sg0, sg1, sout0, sout1):
    sid = lax.axis_index("s")
    wid = sid * NC + lax.axis_index("c")
    tmp_i, tmp_o = (ti0, ti1), (to0, to1)
    idx_f, out_f = (if0, if1), (of0, of1)
    sin, sg, sout = (sin0, sin1), (sg0, sg1), (sout0, sout1)
    r00 = wid * ROWS_W

    def in_cp(k, b):
        return pltpu.make_async_copy(
            idx_hbm.at[pl.ds(r00 + k * RCHUNK, RCHUNK), :], tmp_i[b], sin[b])

    def out_cp(k, b):
        return pltpu.make_async_copy(
            tmp_o[b], out_hbm.at[pl.ds(r00 + k * RCHUNK, RCHUNK), :], sout[b])

    def gather_cp(b):
        return pltpu.make_async_copy(u_sp.at[idx_f[b]], out_f[b], sg[b])

    def pack(b):
        for r in range(RCHUNK):
            for c in MCOLS:
                idx_f[b][pl.ds(r * RSTRIDE + c, 16)] = tmp_i[b][r, pl.ds(c, 16)]

    def unpack(b):
        for r in range(RCHUNK):
            for c in MCOLS:
                tmp_o[b][r, pl.ds(c, 16)] = out_f[b][pl.ds(r * RSTRIDE + c, 16)]

    for b in (0, 1):
        for r in range(RCHUNK):
            base = lax.iota(jnp.int32, 16) * 8 + (r * 64 + b * 2048)
            idx_f[b][pl.ds(r * RSTRIDE + 192, 16)] = base

    for j in range((PIECES + NS - 1) // NS):
        piece = sid + NS * j

        @pl.when(piece < PIECES)
        def _stage():
            off = piece * PIECE
            pltpu.sync_copy(u_hbm.at[pl.ds(off, PIECE)], bounce)
            pltpu.sync_copy(bounce, u_sp.at[pl.ds(off, PIECE)])

    plsc.subcore_barrier()

    in_cp(0, 0).start()

    @pl.loop(0, RCHUNKS // 2)
    def _pair(j):
        for half in (0, 1):
            k = 2 * j + half
            b = half
            in_cp(k, b).wait()
            pack(b)
            gather_cp(b).start()

            @pl.when(k + 1 < RCHUNKS)
            def _prefetch():
                in_cp(k + 1, 1 - b).start()

            @pl.when(k >= 1)
            def _phase2():
                bb = 1 - b
                gather_cp(bb).wait()

                @pl.when(k >= 3)
                def _drain_out():
                    out_cp(k - 3, bb).wait()

                unpack(bb)
                out_cp(k - 1, bb).start()

    gather_cp(1).wait()
    out_cp(RCHUNKS - 3, 1).wait()
    unpack(1)
    out_cp(RCHUNKS - 1, 1).start()
    out_cp(RCHUNKS - 2, 0).wait()
    out_cp(RCHUNKS - 1, 1).wait()


def kernel(idx, u):
    return _gather_sc(idx, u)
